# baseline (speedup 1.0000x reference)
import functools

import jax
import jax.numpy as jnp
from jax import lax
from jax.experimental import pallas as pl
from jax.experimental.pallas import tpu as pltpu
from jax.experimental.pallas import tpu_sc as plsc

LANES = 16
NUM_CORES = 2
NUM_SUBCORES = 16
NUM_WORKERS = NUM_CORES * NUM_SUBCORES
IDX_CHUNK = 128
PACK = 4

_TAKE_DNUMS = lax.GatherDimensionNumbers(
    offset_dims=(), collapsed_slice_dims=(0,), start_index_map=(0,))


def _take16(x, idx):
    return lax.gather(x, idx[:, None], _TAKE_DNUMS, slice_sizes=(1,),
                      mode=lax.GatherScatterMode.PROMISE_IN_BOUNDS)


@functools.lru_cache(maxsize=None)
def _build(batch: int, dim: int):
    b_per_w = batch // NUM_WORKERS
    n_chunks = b_per_w // IDX_CHUNK
    groups_per_chunk = IDX_CHUNK // LANES
    wide = PACK * dim
    vecs_per_slice = wide // LANES

    mesh = plsc.VectorSubcoreMesh(core_axis_name="c", subcore_axis_name="s")

    @functools.partial(
        pl.kernel,
        mesh=mesh,
        out_type=jax.ShapeDtypeStruct((batch,), jnp.float32),
        scratch_types=[
            pltpu.VMEM((n_chunks, IDX_CHUNK), jnp.int32),
            pltpu.VMEM((n_chunks, IDX_CHUNK), jnp.int32),
            pltpu.VMEM((n_chunks, IDX_CHUNK), jnp.float32),
            pltpu.VMEM((n_chunks, IDX_CHUNK), jnp.float32),
            pltpu.VMEM((n_chunks, IDX_CHUNK), jnp.float32),
            pltpu.VMEM((n_chunks, IDX_CHUNK), jnp.float32),
            pltpu.VMEM((IDX_CHUNK, wide), jnp.float32),
            pltpu.VMEM((IDX_CHUNK, wide), jnp.float32),
            pltpu.VMEM((b_per_w,), jnp.float32),
            pltpu.SemaphoreType.DMA,
        ],
    )
    def sc_kernel(uids_hbm, cids_hbm, umat_hbm, imat_hbm, out_hbm,
                  uidx_v, cidx_v, uw0_v, uw1_v, cw0_v, cw1_v,
                  ustage_v, cstage_v, out_v, sem):
        wid = lax.axis_index("s") * NUM_CORES + lax.axis_index("c")
        base = wid * b_per_w

        for j in range(n_chunks):
            pltpu.sync_copy(uids_hbm.at[pl.ds(base + j * IDX_CHUNK, IDX_CHUNK)],
                            uidx_v.at[j])
            pltpu.sync_copy(cids_hbm.at[pl.ds(base + j * IDX_CHUNK, IDX_CHUNK)],
                            cidx_v.at[j])
        ones_i = jnp.full((LANES,), 1, jnp.int32)
        for j in range(n_chunks):
            for k in range(IDX_CHUNK // LANES):
                sl = pl.ds(k * LANES, LANES)
                uid = uidx_v[j, sl]
                cid = cidx_v[j, sl]
                uw0_v[j, sl] = (uid & ones_i).astype(jnp.float32)
                uw1_v[j, sl] = (lax.shift_right_logical(uid, 1)
                                & ones_i).astype(jnp.float32)
                cw0_v[j, sl] = (cid & ones_i).astype(jnp.float32)
                cw1_v[j, sl] = (lax.shift_right_logical(cid, 1)
                                & ones_i).astype(jnp.float32)
                uidx_v[j, sl] = lax.shift_right_logical(uid, 2)
                cidx_v[j, sl] = lax.shift_right_logical(cid, 2)

        lane_iota = lax.iota(jnp.int32, LANES)
        perms = [lane_iota ^ d for d in (1, 2, 4, 8)]
        lane_eq = [lane_iota == r for r in range(LANES)]
        bcast = [jnp.full((LANES,), r, jnp.int32) for r in range(LANES)]
        zeros = jnp.zeros((LANES,), jnp.float32)

        def select_row(stage_ref, row, w0, w1):
            v = [stage_ref[row, pl.ds(t * LANES, LANES)]
                 for t in range(vecs_per_slice)]
            lo0 = v[0] + (v[2] - v[0]) * w0
            lo1 = v[1] + (v[3] - v[1]) * w0
            hi0 = v[4] + (v[6] - v[4]) * w0
            hi1 = v[5] + (v[7] - v[5]) * w0
            return lo0 + (hi0 - lo0) * w1, lo1 + (hi1 - lo1) * w1

        for j in range(n_chunks):
            cu = pltpu.async_copy(umat_hbm.at[uidx_v.at[j]], ustage_v, sem)
            cc = pltpu.async_copy(imat_hbm.at[cidx_v.at[j]], cstage_v, sem)
            cu.wait()
            cc.wait()

            def group_body(g, _):
                row0 = g * LANES
                uw0 = uw0_v[j, pl.ds(row0, LANES)]
                uw1 = uw1_v[j, pl.ds(row0, LANES)]
                cw0 = cw0_v[j, pl.ds(row0, LANES)]
                cw1 = cw1_v[j, pl.ds(row0, LANES)]
                o = zeros
                for r in range(LANES):
                    u0, u1 = select_row(ustage_v, row0 + r,
                                        _take16(uw0, bcast[r]),
                                        _take16(uw1, bcast[r]))
                    c0, c1 = select_row(cstage_v, row0 + r,
                                        _take16(cw0, bcast[r]),
                                        _take16(cw1, bcast[r]))
                    s = u0 * c0 + u1 * c1
                    for perm in perms:
                        s = s + _take16(s, perm)
                    o = jnp.where(lane_eq[r], s, o)
                out_v[pl.ds(j * IDX_CHUNK + row0, LANES)] = (
                    1.0 / (1.0 + jnp.exp(-o)))
                return 0

            lax.fori_loop(0, groups_per_chunk, group_body, 0)

        pltpu.sync_copy(out_v, out_hbm.at[pl.ds(base, b_per_w)])

    return sc_kernel


def kernel(user_ids, content_ids, user_matrix, item_matrix):
    batch = user_ids.shape[0]
    dim = user_matrix.shape[1]
    wide = PACK * dim
    umat = user_matrix.reshape(user_matrix.shape[0] // PACK, wide)
    imat = item_matrix.reshape(item_matrix.shape[0] // PACK, wide)
    return _build(batch, dim)(user_ids, content_ids, umat, imat)


# speedup vs baseline: 1.7188x; 1.7188x over previous
import functools

import jax
import jax.numpy as jnp
from jax import lax
from jax.experimental import pallas as pl
from jax.experimental.pallas import tpu as pltpu
from jax.experimental.pallas import tpu_sc as plsc

LANES = 16
NUM_CORES = 2
NUM_SUBCORES = 16
NUM_WORKERS = NUM_CORES * NUM_SUBCORES
IDX_CHUNK = 128
PACK = 4
COL_BLOCK = 2048

_TAKE_DNUMS = lax.GatherDimensionNumbers(
    offset_dims=(), collapsed_slice_dims=(0,), start_index_map=(0,))


def _take16(x, idx):
    return lax.gather(x, idx[:, None], _TAKE_DNUMS, slice_sizes=(1,),
                      mode=lax.GatherScatterMode.PROMISE_IN_BOUNDS)


def _stride_for(rows: int) -> int:
    per = (rows + PACK - 1) // PACK
    return ((per + COL_BLOCK - 1) // COL_BLOCK) * COL_BLOCK


@functools.lru_cache(maxsize=None)
def _tc_repack(dim: int, rows: int):
    stride = _stride_for(rows)
    blocks_per_region = stride // COL_BLOCK
    max_block = (rows + COL_BLOCK - 1) // COL_BLOCK - 1
    wide = PACK * dim

    def body(*refs):
        xs, y_ref = refs[:PACK], refs[PACK]
        y_ref[...] = jnp.concatenate([x[...].T for x in xs], axis=1)

    def make_map(a):
        return lambda j: (0, jnp.minimum(a * blocks_per_region + j, max_block))

    return pl.pallas_call(
        body,
        grid=(blocks_per_region,),
        in_specs=[pl.BlockSpec((dim, COL_BLOCK), make_map(a))
                  for a in range(PACK)],
        out_specs=pl.BlockSpec((COL_BLOCK, wide), lambda j: (j, 0)),
        out_shape=jax.ShapeDtypeStruct((stride, wide), jnp.float32),
    )


@functools.lru_cache(maxsize=None)
def _build(batch: int, dim: int, ustride: int, istride: int):
    b_per_w = batch // NUM_WORKERS
    n_chunks = b_per_w // IDX_CHUNK
    groups_per_chunk = IDX_CHUNK // LANES
    wide = PACK * dim
    vecs_per_slice = wide // LANES

    mesh = plsc.VectorSubcoreMesh(core_axis_name="c", subcore_axis_name="s")

    @functools.partial(
        pl.kernel,
        mesh=mesh,
        out_type=jax.ShapeDtypeStruct((batch,), jnp.float32),
        scratch_types=[
            pltpu.VMEM((n_chunks, IDX_CHUNK), jnp.int32),
            pltpu.VMEM((n_chunks, IDX_CHUNK), jnp.int32),
            pltpu.VMEM((n_chunks, IDX_CHUNK), jnp.float32),
            pltpu.VMEM((n_chunks, IDX_CHUNK), jnp.float32),
            pltpu.VMEM((n_chunks, IDX_CHUNK), jnp.float32),
            pltpu.VMEM((n_chunks, IDX_CHUNK), jnp.float32),
            pltpu.VMEM((IDX_CHUNK, wide), jnp.float32),
            pltpu.VMEM((IDX_CHUNK, wide), jnp.float32),
            pltpu.VMEM((b_per_w,), jnp.float32),
            pltpu.SemaphoreType.DMA,
        ],
    )
    def sc_kernel(uids_hbm, cids_hbm, umat_hbm, imat_hbm, out_hbm,
                  uidx_v, cidx_v, uw0_v, uw1_v, cw0_v, cw1_v,
                  ustage_v, cstage_v, out_v, sem):
        wid = lax.axis_index("s") * NUM_CORES + lax.axis_index("c")
        base = wid * b_per_w

        for j in range(n_chunks):
            pltpu.sync_copy(uids_hbm.at[pl.ds(base + j * IDX_CHUNK, IDX_CHUNK)],
                            uidx_v.at[j])
            pltpu.sync_copy(cids_hbm.at[pl.ds(base + j * IDX_CHUNK, IDX_CHUNK)],
                            cidx_v.at[j])

        ones_i = jnp.full((LANES,), 1, jnp.int32)

        def region_split(rid, stride):
            a = jnp.full((LANES,), 3, jnp.int32)
            for k in (1, 2, 3):
                a = a - lax.shift_right_logical(rid - (k * stride), 31)
            w = rid - a * stride
            return a, w

        for j in range(n_chunks):
            for k in range(IDX_CHUNK // LANES):
                sl = pl.ds(k * LANES, LANES)
                uid = uidx_v[j, sl]
                cid = cidx_v[j, sl]
                ua, uw = region_split(uid, ustride)
                ca, cw = region_split(cid, istride)
                uw0_v[j, sl] = (ua & ones_i).astype(jnp.float32)
                uw1_v[j, sl] = lax.shift_right_logical(ua, 1).astype(jnp.float32)
                cw0_v[j, sl] = (ca & ones_i).astype(jnp.float32)
                cw1_v[j, sl] = lax.shift_right_logical(ca, 1).astype(jnp.float32)
                uidx_v[j, sl] = uw
                cidx_v[j, sl] = cw

        lane_iota = lax.iota(jnp.int32, LANES)
        perms = [lane_iota ^ d for d in (1, 2, 4, 8)]
        lane_eq = [lane_iota == r for r in range(LANES)]
        bcast = [jnp.full((LANES,), r, jnp.int32) for r in range(LANES)]
        zeros = jnp.zeros((LANES,), jnp.float32)

        def select_row(stage_ref, row, w0, w1):
            v = [stage_ref[row, pl.ds(t * LANES, LANES)]
                 for t in range(vecs_per_slice)]
            lo0 = v[0] + (v[2] - v[0]) * w0
            lo1 = v[1] + (v[3] - v[1]) * w0
            hi0 = v[4] + (v[6] - v[4]) * w0
            hi1 = v[5] + (v[7] - v[5]) * w0
            return lo0 + (hi0 - lo0) * w1, lo1 + (hi1 - lo1) * w1

        for j in range(n_chunks):
            cu = pltpu.async_copy(umat_hbm.at[uidx_v.at[j]], ustage_v, sem)
            cc = pltpu.async_copy(imat_hbm.at[cidx_v.at[j]], cstage_v, sem)
            cu.wait()
            cc.wait()

            def group_body(g, _):
                row0 = g * LANES
                uw0 = uw0_v[j, pl.ds(row0, LANES)]
                uw1 = uw1_v[j, pl.ds(row0, LANES)]
                cw0 = cw0_v[j, pl.ds(row0, LANES)]
                cw1 = cw1_v[j, pl.ds(row0, LANES)]
                o = zeros
                for r in range(LANES):
                    u0, u1 = select_row(ustage_v, row0 + r,
                                        _take16(uw0, bcast[r]),
                                        _take16(uw1, bcast[r]))
                    c0, c1 = select_row(cstage_v, row0 + r,
                                        _take16(cw0, bcast[r]),
                                        _take16(cw1, bcast[r]))
                    s = u0 * c0 + u1 * c1
                    for perm in perms:
                        s = s + _take16(s, perm)
                    o = jnp.where(lane_eq[r], s, o)
                out_v[pl.ds(j * IDX_CHUNK + row0, LANES)] = (
                    1.0 / (1.0 + jnp.exp(-o)))
                return 0

            lax.fori_loop(0, groups_per_chunk, group_body, 0)

        pltpu.sync_copy(out_v, out_hbm.at[pl.ds(base, b_per_w)])

    return sc_kernel


def kernel(user_ids, content_ids, user_matrix, item_matrix):
    batch = user_ids.shape[0]
    dim = user_matrix.shape[1]
    u_rows, i_rows = user_matrix.shape[0], item_matrix.shape[0]
    umat_t, imat_t = user_matrix.T, item_matrix.T
    umat = _tc_repack(dim, u_rows)(*([umat_t] * PACK))
    imat = _tc_repack(dim, i_rows)(*([imat_t] * PACK))
    return _build(batch, dim, _stride_for(u_rows),
                  _stride_for(i_rows))(user_ids, content_ids, umat, imat)


# speedup vs baseline: 3.1730x; 1.8460x over previous
import functools

import jax
import jax.numpy as jnp
from jax import lax
from jax.experimental import pallas as pl
from jax.experimental.pallas import tpu as pltpu
from jax.experimental.pallas import tpu_sc as plsc

LANES = 16
NUM_CORES = 2
NUM_SUBCORES = 16
NUM_WORKERS = NUM_CORES * NUM_SUBCORES
IDX_CHUNK = 128
PACK = 4
COL_BLOCK = 8192

_TAKE_DNUMS = lax.GatherDimensionNumbers(
    offset_dims=(), collapsed_slice_dims=(0,), start_index_map=(0,))


def _take16(x, idx):
    return lax.gather(x, idx[:, None], _TAKE_DNUMS, slice_sizes=(1,),
                      mode=lax.GatherScatterMode.PROMISE_IN_BOUNDS)


def _stride_for(rows: int) -> int:
    per = (rows + PACK - 1) // PACK
    return ((per + COL_BLOCK - 1) // COL_BLOCK) * COL_BLOCK


@functools.lru_cache(maxsize=None)
def _tc_repack(dim: int, rows: int):
    stride = _stride_for(rows)
    blocks_per_region = stride // COL_BLOCK
    max_block = (rows + COL_BLOCK - 1) // COL_BLOCK - 1
    wide = PACK * dim

    def body(*refs):
        xs, y_ref = refs[:PACK], refs[PACK]
        eye = (lax.broadcasted_iota(jnp.int32, (wide, wide), 0)
               == lax.broadcasted_iota(jnp.int32, (wide, wide), 1)
               ).astype(jnp.float32)
        x = jnp.concatenate([x[...] for x in xs], axis=0)
        y_ref[...] = lax.dot_general(
            x, eye, (((0,), (0,)), ((), ())),
            precision=lax.Precision.HIGHEST,
            preferred_element_type=jnp.float32)

    def make_map(a):
        return lambda j: (0, jnp.minimum(a * blocks_per_region + j, max_block))

    return pl.pallas_call(
        body,
        grid=(blocks_per_region,),
        in_specs=[pl.BlockSpec((dim, COL_BLOCK), make_map(a))
                  for a in range(PACK)],
        out_specs=pl.BlockSpec((COL_BLOCK, wide), lambda j: (j, 0)),
        out_shape=jax.ShapeDtypeStruct((stride, wide), jnp.float32),
    )


@functools.lru_cache(maxsize=None)
def _build(batch: int, dim: int, ustride: int, istride: int):
    b_per_w = batch // NUM_WORKERS
    n_chunks = b_per_w // IDX_CHUNK
    groups_per_chunk = IDX_CHUNK // LANES
    wide = PACK * dim
    vecs_per_slice = wide // LANES

    mesh = plsc.VectorSubcoreMesh(core_axis_name="c", subcore_axis_name="s")

    @functools.partial(
        pl.kernel,
        mesh=mesh,
        out_type=jax.ShapeDtypeStruct((batch,), jnp.float32),
        scratch_types=[
            pltpu.VMEM((n_chunks, IDX_CHUNK), jnp.int32),
            pltpu.VMEM((n_chunks, IDX_CHUNK), jnp.int32),
            pltpu.VMEM((n_chunks, IDX_CHUNK), jnp.float32),
            pltpu.VMEM((n_chunks, IDX_CHUNK), jnp.float32),
            pltpu.VMEM((n_chunks, IDX_CHUNK), jnp.float32),
            pltpu.VMEM((n_chunks, IDX_CHUNK), jnp.float32),
            pltpu.VMEM((IDX_CHUNK, wide), jnp.float32),
            pltpu.VMEM((IDX_CHUNK, wide), jnp.float32),
            pltpu.VMEM((b_per_w,), jnp.float32),
            pltpu.SemaphoreType.DMA,
        ],
    )
    def sc_kernel(uids_hbm, cids_hbm, umat_hbm, imat_hbm, out_hbm,
                  uidx_v, cidx_v, uw0_v, uw1_v, cw0_v, cw1_v,
                  ustage_v, cstage_v, out_v, sem):
        wid = lax.axis_index("s") * NUM_CORES + lax.axis_index("c")
        base = wid * b_per_w

        for j in range(n_chunks):
            pltpu.sync_copy(uids_hbm.at[pl.ds(base + j * IDX_CHUNK, IDX_CHUNK)],
                            uidx_v.at[j])
            pltpu.sync_copy(cids_hbm.at[pl.ds(base + j * IDX_CHUNK, IDX_CHUNK)],
                            cidx_v.at[j])

        ones_i = jnp.full((LANES,), 1, jnp.int32)

        def region_split(rid, stride):
            a = jnp.full((LANES,), 3, jnp.int32)
            for k in (1, 2, 3):
                a = a - lax.shift_right_logical(rid - (k * stride), 31)
            w = rid - a * stride
            return a, w

        for j in range(n_chunks):
            for k in range(IDX_CHUNK // LANES):
                sl = pl.ds(k * LANES, LANES)
                uid = uidx_v[j, sl]
                cid = cidx_v[j, sl]
                ua, uw = region_split(uid, ustride)
                ca, cw = region_split(cid, istride)
                uw0_v[j, sl] = (ua & ones_i).astype(jnp.float32)
                uw1_v[j, sl] = lax.shift_right_logical(ua, 1).astype(jnp.float32)
                cw0_v[j, sl] = (ca & ones_i).astype(jnp.float32)
                cw1_v[j, sl] = lax.shift_right_logical(ca, 1).astype(jnp.float32)
                uidx_v[j, sl] = uw
                cidx_v[j, sl] = cw

        lane_iota = lax.iota(jnp.int32, LANES)
        perms = [lane_iota ^ d for d in (1, 2, 4, 8)]
        lane_eq = [lane_iota == r for r in range(LANES)]
        bcast = [jnp.full((LANES,), r, jnp.int32) for r in range(LANES)]
        zeros = jnp.zeros((LANES,), jnp.float32)

        def select_row(stage_ref, row, w0, w1):
            v = [stage_ref[row, pl.ds(t * LANES, LANES)]
                 for t in range(vecs_per_slice)]
            lo0 = v[0] + (v[2] - v[0]) * w0
            lo1 = v[1] + (v[3] - v[1]) * w0
            hi0 = v[4] + (v[6] - v[4]) * w0
            hi1 = v[5] + (v[7] - v[5]) * w0
            return lo0 + (hi0 - lo0) * w1, lo1 + (hi1 - lo1) * w1

        for j in range(n_chunks):
            cu = pltpu.async_copy(umat_hbm.at[uidx_v.at[j]], ustage_v, sem)
            cc = pltpu.async_copy(imat_hbm.at[cidx_v.at[j]], cstage_v, sem)
            cu.wait()
            cc.wait()

            def group_body(g, _):
                row0 = g * LANES
                uw0 = uw0_v[j, pl.ds(row0, LANES)]
                uw1 = uw1_v[j, pl.ds(row0, LANES)]
                cw0 = cw0_v[j, pl.ds(row0, LANES)]
                cw1 = cw1_v[j, pl.ds(row0, LANES)]
                o = zeros
                for r in range(LANES):
                    u0, u1 = select_row(ustage_v, row0 + r,
                                        _take16(uw0, bcast[r]),
                                        _take16(uw1, bcast[r]))
                    c0, c1 = select_row(cstage_v, row0 + r,
                                        _take16(cw0, bcast[r]),
                                        _take16(cw1, bcast[r]))
                    s = u0 * c0 + u1 * c1
                    for perm in perms:
                        s = s + _take16(s, perm)
                    o = jnp.where(lane_eq[r], s, o)
                out_v[pl.ds(j * IDX_CHUNK + row0, LANES)] = (
                    1.0 / (1.0 + jnp.exp(-o)))
                return 0

            lax.fori_loop(0, groups_per_chunk, group_body, 0)

        pltpu.sync_copy(out_v, out_hbm.at[pl.ds(base, b_per_w)])

    return sc_kernel


def kernel(user_ids, content_ids, user_matrix, item_matrix):
    batch = user_ids.shape[0]
    dim = user_matrix.shape[1]
    u_rows, i_rows = user_matrix.shape[0], item_matrix.shape[0]
    umat_t, imat_t = user_matrix.T, item_matrix.T
    umat = _tc_repack(dim, u_rows)(*([umat_t] * PACK))
    imat = _tc_repack(dim, i_rows)(*([imat_t] * PACK))
    return _build(batch, dim, _stride_for(u_rows),
                  _stride_for(i_rows))(user_ids, content_ids, umat, imat)


# speedup vs baseline: 4.2508x; 1.3397x over previous
import functools

import jax
import jax.numpy as jnp
from jax import lax
from jax.experimental import pallas as pl
from jax.experimental.pallas import tpu as pltpu
from jax.experimental.pallas import tpu_sc as plsc

LANES = 16
NUM_CORES = 2
NUM_SUBCORES = 16
NUM_WORKERS = NUM_CORES * NUM_SUBCORES
IDX_CHUNK = 128
PACK = 4
COL_BLOCK = 8192

_TAKE_DNUMS = lax.GatherDimensionNumbers(
    offset_dims=(), collapsed_slice_dims=(0,), start_index_map=(0,))


def _take16(x, idx):
    return lax.gather(x, idx[:, None], _TAKE_DNUMS, slice_sizes=(1,),
                      mode=lax.GatherScatterMode.PROMISE_IN_BOUNDS)


def _stride_for(rows: int) -> int:
    per = (rows + PACK - 1) // PACK
    return ((per + COL_BLOCK - 1) // COL_BLOCK) * COL_BLOCK


@functools.lru_cache(maxsize=None)
def _tc_repack(dim: int, rows: int):
    stride = _stride_for(rows)
    blocks_per_region = stride // COL_BLOCK
    max_block = (rows + COL_BLOCK - 1) // COL_BLOCK - 1
    wide = PACK * dim

    def body(*refs):
        xs, y_ref = refs[:PACK], refs[PACK]
        eye = (lax.broadcasted_iota(jnp.int32, (wide, wide), 0)
               == lax.broadcasted_iota(jnp.int32, (wide, wide), 1)
               ).astype(jnp.bfloat16)
        x = jnp.concatenate([x[...] for x in xs], axis=0)
        hi = x.astype(jnp.bfloat16)
        lo = (x - hi.astype(jnp.float32)).astype(jnp.bfloat16)
        dn = (((0,), (0,)), ((), ()))
        y_ref[...] = (
            lax.dot_general(hi, eye, dn,
                            preferred_element_type=jnp.float32)
            + lax.dot_general(lo, eye, dn,
                              preferred_element_type=jnp.float32))

    def make_map(a):
        return lambda j: (0, jnp.minimum(a * blocks_per_region + j, max_block))

    return pl.pallas_call(
        body,
        grid=(blocks_per_region,),
        in_specs=[pl.BlockSpec((dim, COL_BLOCK), make_map(a))
                  for a in range(PACK)],
        out_specs=pl.BlockSpec((COL_BLOCK, wide), lambda j: (j, 0)),
        out_shape=jax.ShapeDtypeStruct((stride, wide), jnp.float32),
    )


@functools.lru_cache(maxsize=None)
def _build(batch: int, dim: int, ustride: int, istride: int):
    b_per_w = batch // NUM_WORKERS
    n_chunks = b_per_w // IDX_CHUNK
    groups_per_chunk = IDX_CHUNK // LANES
    wide = PACK * dim
    vecs_per_slice = wide // LANES

    mesh = plsc.VectorSubcoreMesh(core_axis_name="c", subcore_axis_name="s")

    @functools.partial(
        pl.kernel,
        mesh=mesh,
        out_type=jax.ShapeDtypeStruct((batch,), jnp.float32),
        scratch_types=[
            pltpu.VMEM((n_chunks, IDX_CHUNK), jnp.int32),
            pltpu.VMEM((n_chunks, IDX_CHUNK), jnp.int32),
            pltpu.VMEM((n_chunks, IDX_CHUNK), jnp.float32),
            pltpu.VMEM((n_chunks, IDX_CHUNK), jnp.float32),
            pltpu.VMEM((n_chunks, IDX_CHUNK), jnp.float32),
            pltpu.VMEM((n_chunks, IDX_CHUNK), jnp.float32),
            pltpu.VMEM((2, IDX_CHUNK, wide), jnp.float32),
            pltpu.VMEM((2, IDX_CHUNK, wide), jnp.float32),
            pltpu.VMEM((b_per_w,), jnp.float32),
            pltpu.SemaphoreType.DMA,
        ],
    )
    def sc_kernel(uids_hbm, cids_hbm, umat_hbm, imat_hbm, out_hbm,
                  uidx_v, cidx_v, uw0_v, uw1_v, cw0_v, cw1_v,
                  ustage_v, cstage_v, out_v, sem):
        wid = lax.axis_index("s") * NUM_CORES + lax.axis_index("c")
        base = wid * b_per_w

        for j in range(n_chunks):
            pltpu.sync_copy(uids_hbm.at[pl.ds(base + j * IDX_CHUNK, IDX_CHUNK)],
                            uidx_v.at[j])
            pltpu.sync_copy(cids_hbm.at[pl.ds(base + j * IDX_CHUNK, IDX_CHUNK)],
                            cidx_v.at[j])

        ones_i = jnp.full((LANES,), 1, jnp.int32)

        def region_split(rid, stride):
            a = jnp.full((LANES,), 3, jnp.int32)
            for k in (1, 2, 3):
                a = a - lax.shift_right_logical(rid - (k * stride), 31)
            w = rid - a * stride
            return a, w

        for j in range(n_chunks):
            for k in range(IDX_CHUNK // LANES):
                sl = pl.ds(k * LANES, LANES)
                uid = uidx_v[j, sl]
                cid = cidx_v[j, sl]
                ua, uw = region_split(uid, ustride)
                ca, cw = region_split(cid, istride)
                uw0_v[j, sl] = (ua & ones_i).astype(jnp.float32)
                uw1_v[j, sl] = lax.shift_right_logical(ua, 1).astype(jnp.float32)
                cw0_v[j, sl] = (ca & ones_i).astype(jnp.float32)
                cw1_v[j, sl] = lax.shift_right_logical(ca, 1).astype(jnp.float32)
                uidx_v[j, sl] = uw
                cidx_v[j, sl] = cw

        lane_iota = lax.iota(jnp.int32, LANES)
        perms = [lane_iota ^ d for d in (1, 2, 4, 8)]
        lane_eq = [lane_iota == r for r in range(LANES)]
        bcast = [jnp.full((LANES,), r, jnp.int32) for r in range(LANES)]
        zeros = jnp.zeros((LANES,), jnp.float32)

        def select_row(stage_ref, row, w0, w1):
            v = [stage_ref[row, pl.ds(t * LANES, LANES)]
                 for t in range(vecs_per_slice)]
            lo0 = v[0] + (v[2] - v[0]) * w0
            lo1 = v[1] + (v[3] - v[1]) * w0
            hi0 = v[4] + (v[6] - v[4]) * w0
            hi1 = v[5] + (v[7] - v[5]) * w0
            return lo0 + (hi0 - lo0) * w1, lo1 + (hi1 - lo1) * w1

        def fire(j):
            b = j % 2
            return (pltpu.async_copy(umat_hbm.at[uidx_v.at[j]],
                                     ustage_v.at[b], sem),
                    pltpu.async_copy(imat_hbm.at[cidx_v.at[j]],
                                     cstage_v.at[b], sem))

        inflight = [fire(0)]
        for j in range(n_chunks):
            if j + 1 < n_chunks:
                inflight.append(fire(j + 1))
            for c in inflight[j]:
                c.wait()
            b = j % 2
            ustage = ustage_v.at[b]
            cstage = cstage_v.at[b]

            def group_body(g, _, j=j, ustage=ustage, cstage=cstage):
                row0 = g * LANES
                uw0 = uw0_v[j, pl.ds(row0, LANES)]
                uw1 = uw1_v[j, pl.ds(row0, LANES)]
                cw0 = cw0_v[j, pl.ds(row0, LANES)]
                cw1 = cw1_v[j, pl.ds(row0, LANES)]
                o = zeros
                for r in range(LANES):
                    u0, u1 = select_row(ustage, row0 + r,
                                        _take16(uw0, bcast[r]),
                                        _take16(uw1, bcast[r]))
                    c0, c1 = select_row(cstage, row0 + r,
                                        _take16(cw0, bcast[r]),
                                        _take16(cw1, bcast[r]))
                    s = u0 * c0 + u1 * c1
                    for perm in perms:
                        s = s + _take16(s, perm)
                    o = jnp.where(lane_eq[r], s, o)
                out_v[pl.ds(j * IDX_CHUNK + row0, LANES)] = (
                    1.0 / (1.0 + jnp.exp(-o)))
                return 0

            lax.fori_loop(0, groups_per_chunk, group_body, 0)

        pltpu.sync_copy(out_v, out_hbm.at[pl.ds(base, b_per_w)])

    return sc_kernel


def kernel(user_ids, content_ids, user_matrix, item_matrix):
    batch = user_ids.shape[0]
    dim = user_matrix.shape[1]
    u_rows, i_rows = user_matrix.shape[0], item_matrix.shape[0]
    umat_t, imat_t = user_matrix.T, item_matrix.T
    umat = _tc_repack(dim, u_rows)(*([umat_t] * PACK))
    imat = _tc_repack(dim, i_rows)(*([imat_t] * PACK))
    return _build(batch, dim, _stride_for(u_rows),
                  _stride_for(i_rows))(user_ids, content_ids, umat, imat)
